# Initial kernel scaffold; baseline (speedup 1.0000x reference)
import jax
import jax.numpy as jnp
from jax.experimental import pallas as pl


def kernel(x, edge_index, batch, W1, asrc1, adst1, b1, g1, be1, W2, asrc2, adst2, b2, g2, be2, Wq, bq, Wr, br):
    raise NotImplementedError("write your pallas kernel here")


# speedup vs baseline: 12.5083x; 12.5083x over previous
import functools

import jax
import jax.numpy as jnp
from jax import lax
from jax.experimental import pallas as pl
from jax.experimental.pallas import tpu as pltpu
from jax.experimental.pallas import tpu_sc as plsc

_N = 10000
_E = 320000
_D = 128
_G = 64

_Q = 32
_TW = 48
_NT = 16
_NW = 32
_EPW = _E // _NW
_EPT = _E // _NT
_K = 80
_NCH = _EPT // _K
_IRW = _EPW // _K
_IRT = _E // _K
_RPT = 1256
_RACC = _NT * _RPT
_NPAD = _RACC // 2
_NZ = 8
_RB = 2000
_NB = _N // _RB


def _p1_body(h_ref, w_ref, av_ref, xw_ref, ssrc_ref, sdst_ref):
    h = h_ref[...]
    xw = jnp.dot(h, w_ref[...], preferred_element_type=jnp.float32)
    s2 = jnp.dot(xw, av_ref[...], preferred_element_type=jnp.float32)
    xw_ref[...] = xw
    ssrc_ref[...] = s2[:, 0:1]
    sdst_ref[...] = s2[:, 1:2]


_p1 = pl.pallas_call(
    _p1_body,
    out_shape=[
        jax.ShapeDtypeStruct((_N, _D), jnp.float32),
        jax.ShapeDtypeStruct((_N, 1), jnp.float32),
        jax.ShapeDtypeStruct((_N, 1), jnp.float32),
    ],
)


def _p2_body(xw_ref, ssrc_ref, t0_ref, t1_ref, t2_ref, t3_ref):
    ssrc_all = ssrc_ref[...]
    cs = jnp.max(ssrc_all)
    i = pl.program_id(0)
    sblk = ssrc_ref[pl.ds(i * _RB, _RB), :]
    xw = xw_ref[...]
    fp = jnp.exp(sblk - cs)
    fm = jnp.exp(0.2 * (sblk - cs))
    z = jnp.zeros((_RB, _TW - _Q - 1), jnp.float32)
    for qi, t_ref in enumerate((t0_ref, t1_ref, t2_ref, t3_ref)):
        xwq = xw[:, qi * _Q:(qi + 1) * _Q]
        rp = jnp.concatenate([fp * xwq, fp, z], axis=1)
        rm = jnp.concatenate([fm * xwq, fm, z], axis=1)
        t_ref[...] = jnp.stack([rp, rm], axis=0)


_p2 = pl.pallas_call(
    _p2_body,
    grid=(_NB,),
    in_specs=[
        pl.BlockSpec((_RB, _D), lambda i: (i, 0)),
        pl.BlockSpec((_N, 1), lambda i: (0, 0)),
    ],
    out_specs=[pl.BlockSpec((2, _RB, _TW), lambda i: (0, i, 0))] * 4,
    out_shape=[jax.ShapeDtypeStruct((2, _N, _TW), jnp.float32)] * 4,
)


def _index_body(src_h, dst_h, ssrc_h, sdst_h, gi_h, si_h,
                srcv, dstv, ssv, sdv, gib, sib):
    cid = lax.axis_index("c")
    sid = lax.axis_index("s")
    w = cid * _NT + sid
    ebase = w * _EPW
    pltpu.sync_copy(src_h.at[pl.ds(ebase, _EPW)], srcv)
    pltpu.sync_copy(dst_h.at[pl.ds(ebase, _EPW)], dstv)
    pltpu.sync_copy(ssrc_h, ssv)
    pltpu.sync_copy(sdst_h, sdv)

    def idx_body(r, carry):
        for j in range(_K // 16):
            sl = pl.ds(r * _K + j * 16, 16)
            s = srcv[sl]
            d = dstv[sl]
            a = plsc.load_gather(ssv, [s]) + plsc.load_gather(sdv, [d])
            neg = a <= 0.0
            gib[r, 0, pl.ds(j * 16, 16)] = s + jnp.where(neg, _N, 0).astype(jnp.int32)
            sib[r, 0, pl.ds(j * 16, 16)] = d + jnp.where(neg, _NPAD, 0).astype(jnp.int32)
        return carry

    lax.fori_loop(0, _IRW, idx_body, 0)
    rbase = w * _IRW
    pltpu.sync_copy(gib, gi_h.at[pl.ds(rbase, _IRW)])
    pltpu.sync_copy(sib, si_h.at[pl.ds(rbase, _IRW)])


_index = functools.partial(
    pl.kernel,
    mesh=plsc.VectorSubcoreMesh(core_axis_name="c", subcore_axis_name="s"),
    compiler_params=pltpu.CompilerParams(needs_layout_passes=False,
                                         use_tc_tiling_on_sc=False),
    out_type=[jax.ShapeDtypeStruct((_IRT, 1, _K), jnp.int32)] * 2,
    scratch_types=[
        pltpu.VMEM((_EPW,), jnp.int32),
        pltpu.VMEM((_EPW,), jnp.int32),
        pltpu.VMEM((_N,), jnp.float32),
        pltpu.VMEM((_N,), jnp.float32),
        pltpu.VMEM((_IRW, 1, _K), jnp.int32),
        pltpu.VMEM((_IRW, 1, _K), jnp.int32),
    ],
)(_index_body)


def _stream_body(gi_h, si_h, t0_h, t1_h, t2_h, t3_h,
                 o0_h, o1_h, o2_h, o3_h,
                 gidx, sidx, gb0, zb, acc):
    cid = lax.axis_index("c")
    sid = lax.axis_index("s")
    ibase = sid * _NCH
    pltpu.sync_copy(gi_h.at[pl.ds(ibase, _NCH)], gidx)
    pltpu.sync_copy(si_h.at[pl.ds(ibase, _NCH)], sidx)

    z16 = jnp.zeros((16,), jnp.float32)

    def zb_body(r, carry):
        for j in range(_TW // 16):
            zb[r, pl.ds(j * 16, 16)] = z16
        return carry

    lax.fori_loop(0, _NZ, zb_body, 0)

    rbase = sid * _RPT

    def zero_stripe():
        def zc_body(k, carry):
            pltpu.sync_copy(zb, acc.at[pl.ds(rbase + k * _NZ, _NZ)])
            return carry

        lax.fori_loop(0, _RPT // _NZ, zc_body, 0)

    def run(t_list, o_list):
        for p in range(2):
            zero_stripe()
            plsc.subcore_barrier()

            def ch_body(c2, carry):
                pltpu.sync_copy(t_list[p].at[gidx.at[c2, 0]], gb0)
                pltpu.sync_copy(gb0, acc.at[sidx.at[c2, 0]], add=True)
                return carry

            lax.fori_loop(0, _NCH, ch_body, 0)
            plsc.subcore_barrier()
            pltpu.sync_copy(acc.at[pl.ds(rbase, _RPT)],
                            o_list[p].at[pl.ds(rbase, _RPT)])
            plsc.subcore_barrier()

    @pl.when(cid == 0)
    def _():
        run((t0_h, t1_h), (o0_h, o1_h))

    @pl.when(cid == 1)
    def _():
        run((t2_h, t3_h), (o2_h, o3_h))


_stream = functools.partial(
    pl.kernel,
    mesh=plsc.VectorSubcoreMesh(core_axis_name="c", subcore_axis_name="s"),
    compiler_params=pltpu.CompilerParams(needs_layout_passes=False,
                                         use_tc_tiling_on_sc=False),
    out_type=[jax.ShapeDtypeStruct((_RACC, _TW), jnp.float32)] * 4,
    scratch_types=[
        pltpu.VMEM((_NCH, 1, _K), jnp.int32),
        pltpu.VMEM((_NCH, 1, _K), jnp.int32),
        pltpu.VMEM((_K, _TW), jnp.float32),
        pltpu.VMEM((_NZ, _TW), jnp.float32),
        pltpu.VMEM_SHARED((_RACC, _TW), jnp.float32),
    ],
)(_stream_body)


def _e1_body(o0_ref, o1_ref, o2_ref, o3_ref, ssrc_ref, sdst_ref, b_ref,
             gat_ref, ps_ref, pq_ref):
    ssrc = ssrc_ref[...]
    sdst_all = sdst_ref[...]
    cs = jnp.max(ssrc)
    cd = jnp.max(sdst_all)
    c = cs + cd
    i = pl.program_id(0)
    sd = sdst_ref[pl.ds(i * _RB, _RB), :]
    fp = jnp.exp(sd - cd)
    fm = jnp.exp(0.2 * (sd - cd) - 0.8 * c)
    combs = []
    for o_ref in (o0_ref, o1_ref, o2_ref, o3_ref):
        o = o_ref[...]
        combs.append(fp * o[0] + fm * o[1])
    numer = jnp.concatenate([cb[:, :_Q] for cb in combs], axis=1)
    den = combs[0][:, _Q:_Q + 1]
    gat = numer / (den + 1e-16) + b_ref[...]
    gat_ref[...] = gat
    ps_ref[...] = jnp.sum(gat, axis=0, keepdims=True).reshape(1, 1, _D)
    pq_ref[...] = jnp.sum(gat * gat, axis=0, keepdims=True).reshape(1, 1, _D)


_e1 = pl.pallas_call(
    _e1_body,
    grid=(_NB,),
    in_specs=[pl.BlockSpec((2, _RB, _TW), lambda i: (0, i, 0))] * 4 + [
        pl.BlockSpec((_N, 1), lambda i: (0, 0)),
        pl.BlockSpec((_N, 1), lambda i: (0, 0)),
        pl.BlockSpec((1, _D), lambda i: (0, 0)),
    ],
    out_specs=[
        pl.BlockSpec((_RB, _D), lambda i: (i, 0)),
        pl.BlockSpec((1, 1, _D), lambda i: (i, 0, 0)),
        pl.BlockSpec((1, 1, _D), lambda i: (i, 0, 0)),
    ],
    out_shape=[
        jax.ShapeDtypeStruct((_N, _D), jnp.float32),
        jax.ShapeDtypeStruct((_NB, 1, _D), jnp.float32),
        jax.ShapeDtypeStruct((_NB, 1, _D), jnp.float32),
    ],
)


def _e2_body(gat_ref, ps_ref, pq_ref, g_ref, be_ref, hprev_ref, out_ref):
    gat = gat_ref[...]
    m = jnp.sum(ps_ref[...], axis=0) / _N
    m2 = jnp.sum(pq_ref[...], axis=0) / _N
    v = m2 - m * m
    bn = (gat - m) / jnp.sqrt(v + 1e-5) * g_ref[...] + be_ref[...]
    out_ref[...] = jnp.maximum(bn, 0.0) + hprev_ref[...]


_e2 = pl.pallas_call(
    _e2_body,
    out_shape=jax.ShapeDtypeStruct((_N, _D), jnp.float32),
)


def _head_body(h_ref, batch_ref, wq_ref, bq_ref, wr_ref, br_ref, out_ref,
               hmax_ref):
    h = h_ref[...]
    batch = batch_ref[...]
    oh = (batch == lax.broadcasted_iota(jnp.int32, (_N, _G), 1)
          ).astype(jnp.float32)
    dn = (((0,), (0,)), ((), ()))
    sums = lax.dot_general(oh, h, dn, preferred_element_type=jnp.float32)
    cnt = lax.dot_general(oh, jnp.ones((_N, 1), jnp.float32), dn,
                          preferred_element_type=jnp.float32)
    mean = sums / jnp.maximum(cnt, 1.0)

    neg_inf = jnp.float32(-jnp.inf)

    def mx_body(g, carry):
        mask = batch == g
        hm = jnp.max(jnp.where(mask, h, neg_inf), axis=0, keepdims=True)
        hmax_ref[pl.ds(g, 1), :] = hm
        return carry

    lax.fori_loop(0, _G, mx_body, 0)
    hmax = hmax_ref[...]
    hmax = jnp.where(jnp.isfinite(hmax), hmax, 0.0)
    pooled = jnp.concatenate([mean, hmax], axis=1)
    hq = jnp.tanh(jnp.dot(pooled, wq_ref[...],
                          preferred_element_type=jnp.float32) + bq_ref[...])
    comb = jnp.concatenate([pooled, hq], axis=1)
    out_ref[...] = jnp.dot(comb, wr_ref[...],
                           preferred_element_type=jnp.float32) + br_ref[...]


_head = pl.pallas_call(
    _head_body,
    out_shape=jax.ShapeDtypeStruct((_G, 10), jnp.float32),
    scratch_shapes=[pltpu.VMEM((_G, _D), jnp.float32)],
)


def kernel(x, edge_index, batch, W1, asrc1, adst1, b1, g1, be1,
           W2, asrc2, adst2, b2, g2, be2, Wq, bq, Wr, br):
    src = edge_index[0]
    dst = edge_index[1]
    h = x
    for (W, asrc, adst, b, g, be) in ((W1, asrc1, adst1, b1, g1, be1),
                                      (W2, asrc2, adst2, b2, g2, be2)):
        av = jnp.stack([asrc, adst], axis=1)
        xw, ssrc, sdst = _p1(h, W, av)
        t0, t1, t2, t3 = _p2(xw, ssrc)
        gi, si = _index(src, dst, ssrc.reshape(_N), sdst.reshape(_N))
        o0, o1, o2, o3 = _stream(gi, si, t0.reshape(2 * _N, _TW),
                                 t1.reshape(2 * _N, _TW),
                                 t2.reshape(2 * _N, _TW),
                                 t3.reshape(2 * _N, _TW))
        gat, ps, pq = _e1(o0.reshape(2, _NPAD, _TW), o1.reshape(2, _NPAD, _TW),
                          o2.reshape(2, _NPAD, _TW), o3.reshape(2, _NPAD, _TW),
                          ssrc, sdst, b.reshape(1, _D))
        h = _e2(gat, ps, pq, g.reshape(1, _D), be.reshape(1, _D), h)
    return _head(h, batch.reshape(_N, 1), Wq, bq.reshape(1, -1),
                 Wr, br.reshape(1, -1))


# speedup vs baseline: 25.1271x; 2.0088x over previous
import functools

import jax
import jax.numpy as jnp
from jax import lax
from jax.experimental import pallas as pl
from jax.experimental.pallas import tpu as pltpu
from jax.experimental.pallas import tpu_sc as plsc

_N = 10000
_E = 320000
_D = 128
_G = 64

_Q = 32
_TW = 48
_NT = 16
_NW = 32
_EPW = _E // _NW
_EPT = _E // _NT
_K = 80
_NCH = _EPT // _K
_IRW = _EPW // _K
_IRT = _E // _K
_RPT = 1256
_RACC = _NT * _RPT
_NPAD = _RACC // 2
_NZ = 8
_RB = 2000
_NB = _N // _RB


def _p1_body(h_ref, w_ref, av_ref, xw_ref, ssrc_ref, sdst_ref):
    h = h_ref[...]
    xw = jnp.dot(h, w_ref[...], preferred_element_type=jnp.float32)
    s2 = jnp.dot(xw, av_ref[...], preferred_element_type=jnp.float32)
    xw_ref[...] = xw
    ssrc_ref[...] = s2[:, 0:1]
    sdst_ref[...] = s2[:, 1:2]


_p1 = pl.pallas_call(
    _p1_body,
    out_shape=[
        jax.ShapeDtypeStruct((_N, _D), jnp.float32),
        jax.ShapeDtypeStruct((_N, 1), jnp.float32),
        jax.ShapeDtypeStruct((_N, 1), jnp.float32),
    ],
)


def _p2_body(xw_ref, ssrc_ref, t0_ref, t1_ref, t2_ref, t3_ref):
    ssrc_all = ssrc_ref[...]
    cs = jnp.max(ssrc_all)
    i = pl.program_id(0)
    sblk = ssrc_ref[pl.ds(i * _RB, _RB), :]
    xw = xw_ref[...]
    fp = jnp.exp(sblk - cs)
    fm = jnp.exp(0.2 * (sblk - cs))
    z = jnp.zeros((_RB, _TW - _Q - 1), jnp.float32)
    for qi, t_ref in enumerate((t0_ref, t1_ref, t2_ref, t3_ref)):
        xwq = xw[:, qi * _Q:(qi + 1) * _Q]
        rp = jnp.concatenate([fp * xwq, fp, z], axis=1)
        rm = jnp.concatenate([fm * xwq, fm, z], axis=1)
        t_ref[...] = jnp.stack([rp, rm], axis=0)


_p2 = pl.pallas_call(
    _p2_body,
    grid=(_NB,),
    in_specs=[
        pl.BlockSpec((_RB, _D), lambda i: (i, 0)),
        pl.BlockSpec((_N, 1), lambda i: (0, 0)),
    ],
    out_specs=[pl.BlockSpec((2, _RB, _TW), lambda i: (0, i, 0))] * 4,
    out_shape=[jax.ShapeDtypeStruct((2, _N, _TW), jnp.float32)] * 4,
)


def _index_body(src_h, dst_h, ssrc_h, sdst_h, gi_h, si_h,
                srcv, dstv, ssv, sdv, gib, sib):
    cid = lax.axis_index("c")
    sid = lax.axis_index("s")
    w = cid * _NT + sid
    ebase = w * _EPW
    pltpu.sync_copy(src_h.at[pl.ds(ebase, _EPW)], srcv)
    pltpu.sync_copy(dst_h.at[pl.ds(ebase, _EPW)], dstv)
    pltpu.sync_copy(ssrc_h, ssv)
    pltpu.sync_copy(sdst_h, sdv)

    def idx_body(r, carry):
        for j in range(_K // 16):
            sl = pl.ds(r * _K + j * 16, 16)
            s = srcv[sl]
            d = dstv[sl]
            a = plsc.load_gather(ssv, [s]) + plsc.load_gather(sdv, [d])
            neg = a <= 0.0
            gib[r, 0, pl.ds(j * 16, 16)] = s + jnp.where(neg, _N, 0).astype(jnp.int32)
            sib[r, 0, pl.ds(j * 16, 16)] = d + jnp.where(neg, _NPAD, 0).astype(jnp.int32)
        return carry

    lax.fori_loop(0, _IRW, idx_body, 0)
    rbase = w * _IRW
    pltpu.sync_copy(gib, gi_h.at[pl.ds(rbase, _IRW)])
    pltpu.sync_copy(sib, si_h.at[pl.ds(rbase, _IRW)])


_index = functools.partial(
    pl.kernel,
    mesh=plsc.VectorSubcoreMesh(core_axis_name="c", subcore_axis_name="s"),
    compiler_params=pltpu.CompilerParams(needs_layout_passes=False,
                                         use_tc_tiling_on_sc=False),
    out_type=[jax.ShapeDtypeStruct((_IRT, 1, _K), jnp.int32)] * 2,
    scratch_types=[
        pltpu.VMEM((_EPW,), jnp.int32),
        pltpu.VMEM((_EPW,), jnp.int32),
        pltpu.VMEM((_N,), jnp.float32),
        pltpu.VMEM((_N,), jnp.float32),
        pltpu.VMEM((_IRW, 1, _K), jnp.int32),
        pltpu.VMEM((_IRW, 1, _K), jnp.int32),
    ],
)(_index_body)


def _stream_body(gi_h, si_h, t0_h, t1_h, t2_h, t3_h,
                 o0_h, o1_h, o2_h, o3_h,
                 gidx, sidx, gb0, gb1, gb2, gb3, gb4, zb, acc,
                 sg0, sg1, sg2, sg3, sg4, ssem):
    gbs = (gb0, gb1, gb2, gb3, gb4)
    sgs = (sg0, sg1, sg2, sg3, sg4)
    cid = lax.axis_index("c")
    sid = lax.axis_index("s")
    ibase = sid * _NCH
    pltpu.sync_copy(gi_h.at[pl.ds(ibase, _NCH)], gidx)
    pltpu.sync_copy(si_h.at[pl.ds(ibase, _NCH)], sidx)

    z16 = jnp.zeros((16,), jnp.float32)

    def zb_body(r, carry):
        for j in range(_TW // 16):
            zb[r, pl.ds(j * 16, 16)] = z16
        return carry

    lax.fori_loop(0, _NZ, zb_body, 0)

    rbase = sid * _RPT

    def zero_stripe():
        def zc_body(k, carry):
            pltpu.sync_copy(zb, acc.at[pl.ds(rbase + k * _NZ, _NZ)])
            return carry

        lax.fori_loop(0, _RPT // _NZ, zc_body, 0)

    _NBUF = 5

    def run(t_list, o_list):
        for p in range(2):
            zero_stripe()
            plsc.subcore_barrier()
            t_h = t_list[p]
            for b in range(_NBUF):
                pltpu.async_copy(t_h.at[gidx.at[b, 0]], gbs[b], sgs[b])

            def ch_body(t, carry):
                for b in range(_NBUF):
                    c2 = t * _NBUF + b
                    pltpu.make_async_copy(t_h.at[gidx.at[c2, 0]],
                                          gbs[b], sgs[b]).wait()
                    hs = pltpu.async_copy(gbs[b], acc.at[sidx.at[c2, 0]],
                                          ssem, add=True)
                    hs.wait()

                    @pl.when(c2 + _NBUF < _NCH)
                    def _():
                        pltpu.async_copy(t_h.at[gidx.at[c2 + _NBUF, 0]],
                                         gbs[b], sgs[b])
                return carry

            lax.fori_loop(0, _NCH // _NBUF, ch_body, 0)
            plsc.subcore_barrier()
            pltpu.sync_copy(acc.at[pl.ds(rbase, _RPT)],
                            o_list[p].at[pl.ds(rbase, _RPT)])
            plsc.subcore_barrier()

    @pl.when(cid == 0)
    def _():
        run((t0_h, t1_h), (o0_h, o1_h))

    @pl.when(cid == 1)
    def _():
        run((t2_h, t3_h), (o2_h, o3_h))


_stream = functools.partial(
    pl.kernel,
    mesh=plsc.VectorSubcoreMesh(core_axis_name="c", subcore_axis_name="s"),
    compiler_params=pltpu.CompilerParams(needs_layout_passes=False,
                                         use_tc_tiling_on_sc=False),
    out_type=[jax.ShapeDtypeStruct((_RACC, _TW), jnp.float32)] * 4,
    scratch_types=[
        pltpu.VMEM((_NCH, 1, _K), jnp.int32),
        pltpu.VMEM((_NCH, 1, _K), jnp.int32),
        pltpu.VMEM((_K, _TW), jnp.float32),
        pltpu.VMEM((_K, _TW), jnp.float32),
        pltpu.VMEM((_K, _TW), jnp.float32),
        pltpu.VMEM((_K, _TW), jnp.float32),
        pltpu.VMEM((_K, _TW), jnp.float32),
        pltpu.VMEM((_NZ, _TW), jnp.float32),
        pltpu.VMEM_SHARED((_RACC, _TW), jnp.float32),
        pltpu.SemaphoreType.DMA,
        pltpu.SemaphoreType.DMA,
        pltpu.SemaphoreType.DMA,
        pltpu.SemaphoreType.DMA,
        pltpu.SemaphoreType.DMA,
        pltpu.SemaphoreType.DMA,
    ],
)(_stream_body)


def _e1_body(o0_ref, o1_ref, o2_ref, o3_ref, ssrc_ref, sdst_ref, b_ref,
             gat_ref, ps_ref, pq_ref):
    ssrc = ssrc_ref[...]
    sdst_all = sdst_ref[...]
    cs = jnp.max(ssrc)
    cd = jnp.max(sdst_all)
    c = cs + cd
    i = pl.program_id(0)
    sd = sdst_ref[pl.ds(i * _RB, _RB), :]
    fp = jnp.exp(sd - cd)
    fm = jnp.exp(0.2 * (sd - cd) - 0.8 * c)
    combs = []
    for o_ref in (o0_ref, o1_ref, o2_ref, o3_ref):
        o = o_ref[...]
        combs.append(fp * o[0] + fm * o[1])
    numer = jnp.concatenate([cb[:, :_Q] for cb in combs], axis=1)
    den = combs[0][:, _Q:_Q + 1]
    gat = numer / (den + 1e-16) + b_ref[...]
    gat_ref[...] = gat
    ps_ref[...] = jnp.sum(gat, axis=0, keepdims=True).reshape(1, 1, _D)
    pq_ref[...] = jnp.sum(gat * gat, axis=0, keepdims=True).reshape(1, 1, _D)


_e1 = pl.pallas_call(
    _e1_body,
    grid=(_NB,),
    in_specs=[pl.BlockSpec((2, _RB, _TW), lambda i: (0, i, 0))] * 4 + [
        pl.BlockSpec((_N, 1), lambda i: (0, 0)),
        pl.BlockSpec((_N, 1), lambda i: (0, 0)),
        pl.BlockSpec((1, _D), lambda i: (0, 0)),
    ],
    out_specs=[
        pl.BlockSpec((_RB, _D), lambda i: (i, 0)),
        pl.BlockSpec((1, 1, _D), lambda i: (i, 0, 0)),
        pl.BlockSpec((1, 1, _D), lambda i: (i, 0, 0)),
    ],
    out_shape=[
        jax.ShapeDtypeStruct((_N, _D), jnp.float32),
        jax.ShapeDtypeStruct((_NB, 1, _D), jnp.float32),
        jax.ShapeDtypeStruct((_NB, 1, _D), jnp.float32),
    ],
)


def _e2_body(gat_ref, ps_ref, pq_ref, g_ref, be_ref, hprev_ref, out_ref):
    gat = gat_ref[...]
    m = jnp.sum(ps_ref[...], axis=0) / _N
    m2 = jnp.sum(pq_ref[...], axis=0) / _N
    v = m2 - m * m
    bn = (gat - m) / jnp.sqrt(v + 1e-5) * g_ref[...] + be_ref[...]
    out_ref[...] = jnp.maximum(bn, 0.0) + hprev_ref[...]


_e2 = pl.pallas_call(
    _e2_body,
    out_shape=jax.ShapeDtypeStruct((_N, _D), jnp.float32),
)


def _head_body(h_ref, batch_ref, wq_ref, bq_ref, wr_ref, br_ref, out_ref,
               hmax_ref):
    h = h_ref[...]
    batch = batch_ref[...]
    oh = (batch == lax.broadcasted_iota(jnp.int32, (_N, _G), 1)
          ).astype(jnp.float32)
    dn = (((0,), (0,)), ((), ()))
    sums = lax.dot_general(oh, h, dn, preferred_element_type=jnp.float32)
    cnt = lax.dot_general(oh, jnp.ones((_N, 1), jnp.float32), dn,
                          preferred_element_type=jnp.float32)
    mean = sums / jnp.maximum(cnt, 1.0)

    neg_inf = jnp.float32(-jnp.inf)

    def mx_body(g, carry):
        mask = batch == g
        hm = jnp.max(jnp.where(mask, h, neg_inf), axis=0, keepdims=True)
        hmax_ref[pl.ds(g, 1), :] = hm
        return carry

    lax.fori_loop(0, _G, mx_body, 0)
    hmax = hmax_ref[...]
    hmax = jnp.where(jnp.isfinite(hmax), hmax, 0.0)
    pooled = jnp.concatenate([mean, hmax], axis=1)
    hq = jnp.tanh(jnp.dot(pooled, wq_ref[...],
                          preferred_element_type=jnp.float32) + bq_ref[...])
    comb = jnp.concatenate([pooled, hq], axis=1)
    out_ref[...] = jnp.dot(comb, wr_ref[...],
                           preferred_element_type=jnp.float32) + br_ref[...]


_head = pl.pallas_call(
    _head_body,
    out_shape=jax.ShapeDtypeStruct((_G, 10), jnp.float32),
    scratch_shapes=[pltpu.VMEM((_G, _D), jnp.float32)],
)


def kernel(x, edge_index, batch, W1, asrc1, adst1, b1, g1, be1,
           W2, asrc2, adst2, b2, g2, be2, Wq, bq, Wr, br):
    src = edge_index[0]
    dst = edge_index[1]
    h = x
    for (W, asrc, adst, b, g, be) in ((W1, asrc1, adst1, b1, g1, be1),
                                      (W2, asrc2, adst2, b2, g2, be2)):
        av = jnp.stack([asrc, adst], axis=1)
        xw, ssrc, sdst = _p1(h, W, av)
        t0, t1, t2, t3 = _p2(xw, ssrc)
        gi, si = _index(src, dst, ssrc.reshape(_N), sdst.reshape(_N))
        o0, o1, o2, o3 = _stream(gi, si, t0.reshape(2 * _N, _TW),
                                 t1.reshape(2 * _N, _TW),
                                 t2.reshape(2 * _N, _TW),
                                 t3.reshape(2 * _N, _TW))
        gat, ps, pq = _e1(o0.reshape(2, _NPAD, _TW), o1.reshape(2, _NPAD, _TW),
                          o2.reshape(2, _NPAD, _TW), o3.reshape(2, _NPAD, _TW),
                          ssrc, sdst, b.reshape(1, _D))
        h = _e2(gat, ps, pq, g.reshape(1, _D), be.reshape(1, _D), h)
    return _head(h, batch.reshape(_N, 1), Wq, bq.reshape(1, -1),
                 Wr, br.reshape(1, -1))
